# baseline (speedup 1.0000x reference)
import functools

import jax
import jax.numpy as jnp
from jax import lax
from jax.experimental import pallas as pl
from jax.experimental.pallas import tpu as pltpu
from jax.experimental.pallas import tpu_sc as plsc

NC = 2
NS = 16
NW = NC * NS
NBUF = 3


def _segment_sum_sc(x, idx4, zeros):
    n, d = x.shape
    _, c_chunks, _, b = idx4.shape
    ic = 2 * NBUF
    rows_per_tile = (n // NS) // 8 * 8
    tail = n - NS * rows_per_tile
    mesh = plsc.VectorSubcoreMesh(core_axis_name="c", subcore_axis_name="s")

    @functools.partial(
        pl.kernel,
        out_type=jax.ShapeDtypeStruct((NC, n, d), jnp.float32),
        mesh=mesh,
        scratch_types=(
            [pltpu.VMEM((ic, 2, b), jnp.int32)]
            + [pltpu.VMEM((b, d), jnp.float32)] * NBUF
            + [pltpu.VMEM_SHARED((n, d), jnp.float32)]
            + [pltpu.SemaphoreType.DMA] * (ic + 2 * NBUF)
        ),
    )
    def seg_kernel(x_hbm, idx_hbm, zeros_hbm, out_hbm, ring, *rest):
        bufs = rest[:NBUF]
        acc_sh = rest[NBUF]
        isem = rest[NBUF + 1:NBUF + 1 + ic]
        gsem = rest[NBUF + 1 + ic:NBUF + 1 + ic + NBUF]
        ssem = rest[NBUF + 1 + ic + NBUF:]
        cid = lax.axis_index("c")
        sid = lax.axis_index("s")
        wid = sid * NC + cid
        my_rows = pl.ds(sid * rows_per_tile, rows_per_tile)
        tail_rows = pl.ds(NS * rows_per_tile, tail)
        for t in range(ic - 1):
            pltpu.async_copy(idx_hbm.at[wid, t], ring.at[t], isem[t])
        for s in range(2):
            pltpu.make_async_copy(
                idx_hbm.at[wid, s], ring.at[s], isem[s]).wait()
            pltpu.async_copy(x_hbm.at[ring.at[s, 0]], bufs[s], gsem[s])
        pltpu.sync_copy(zeros_hbm.at[my_rows], acc_sh.at[my_rows])
        if tail:
            @pl.when(sid == NS - 1)
            def _zero_tail():
                pltpu.sync_copy(zeros_hbm.at[tail_rows], acc_sh.at[tail_rows])
        plsc.subcore_barrier()

        def emit_chunk(c, k):
            s = k % NBUF
            s2 = (k + 2) % NBUF
            k2 = (k + 2) % ic
            pltpu.make_async_copy(
                x_hbm.at[ring.at[k, 0]], bufs[s], gsem[s]).wait()
            pltpu.async_copy(
                bufs[s], acc_sh.at[ring.at[k, 1]], ssem[s], add=True)

            @pl.when(c >= 1)
            def _drain_prev():
                pltpu.make_async_copy(
                    bufs[s2], acc_sh.at[ring.at[k, 1]], ssem[s2]).wait()

            @pl.when(c + (ic - 1) < c_chunks)
            def _refill_idx():
                pltpu.async_copy(idx_hbm.at[wid, c + (ic - 1)],
                                 ring.at[(k + ic - 1) % ic],
                                 isem[(k + ic - 1) % ic])

            @pl.when(c + 2 < c_chunks)
            def _refill_rows():
                pltpu.make_async_copy(
                    idx_hbm.at[wid, c + 2], ring.at[k2], isem[k2]).wait()
                pltpu.async_copy(x_hbm.at[ring.at[k2, 0]], bufs[s2], gsem[s2])

        def body(g, carry):
            for k in range(ic):
                emit_chunk(g * ic + k, k)
            return carry

        n_groups = c_chunks // ic
        lax.fori_loop(0, n_groups, body, 0)
        for k in range(c_chunks - n_groups * ic):
            emit_chunk(n_groups * ic + k, k)
        s_last = (c_chunks - 1) % NBUF
        pltpu.make_async_copy(
            bufs[s_last], acc_sh.at[ring.at[0, 1]], ssem[s_last]).wait()
        plsc.subcore_barrier()
        pltpu.sync_copy(acc_sh.at[my_rows], out_hbm.at[cid, my_rows])
        if tail:
            @pl.when(sid == NS - 1)
            def _out_tail():
                pltpu.sync_copy(acc_sh.at[tail_rows], out_hbm.at[cid, tail_rows])

    return seg_kernel(x, idx4, zeros)


def _dense_tc(partials, w, bias, relu):
    _, n, d = partials.shape
    d_out = w.shape[1]
    bn = 5000

    def body(p_ref, w_ref, b_ref, o_ref):
        a = p_ref[0] + p_ref[1]
        y = jnp.dot(a, w_ref[...], preferred_element_type=jnp.float32)
        y = y + b_ref[...]
        if relu:
            y = jnp.maximum(y, 0.0)
        o_ref[...] = y

    return pl.pallas_call(
        body,
        grid=(n // bn,),
        in_specs=[
            pl.BlockSpec((NC, bn, d), lambda i: (0, i, 0)),
            pl.BlockSpec((d, d_out), lambda i: (0, 0)),
            pl.BlockSpec((1, d_out), lambda i: (0, 0)),
        ],
        out_specs=pl.BlockSpec((bn, d_out), lambda i: (i, 0)),
        out_shape=jax.ShapeDtypeStruct((n, d_out), jnp.float32),
    )(partials, w, bias.reshape(1, -1))


def kernel(features, edge_index, W0, b0, W1, b1):
    n, d_in = features.shape
    e = edge_index.shape[1]
    e_per_w = e // NW
    b = 100
    c_chunks = e_per_w // b
    idx4 = jnp.stack(
        [edge_index[0].reshape(NW, c_chunks, b),
         edge_index[1].reshape(NW, c_chunks, b)], axis=2)
    zeros = jnp.zeros((n, d_in), jnp.float32)

    p0 = _segment_sum_sc(features, idx4, zeros)
    h = _dense_tc(p0, W0, b0, relu=True)
    p1 = _segment_sum_sc(h, idx4, zeros)
    return _dense_tc(p1, W1, b1, relu=False)


# speedup vs baseline: 1.0437x; 1.0437x over previous
import functools

import jax
import jax.numpy as jnp
from jax import lax
from jax.experimental import pallas as pl
from jax.experimental.pallas import tpu as pltpu
from jax.experimental.pallas import tpu_sc as plsc

NC = 2
NS = 16
NW = NC * NS
NBUF = 3


def _segment_sum_sc(x, src3, dst3, zeros):
    n, d = x.shape
    _, c_chunks, b = src3.shape
    ic = 2 * NBUF
    rows_per_tile = (n // NS) // 8 * 8
    tail = n - NS * rows_per_tile
    mesh = plsc.VectorSubcoreMesh(core_axis_name="c", subcore_axis_name="s")

    @functools.partial(
        pl.kernel,
        out_type=jax.ShapeDtypeStruct((NC, n, d), jnp.float32),
        mesh=mesh,
        scratch_types=(
            [
                pltpu.VMEM((ic, b), jnp.int32),
                pltpu.VMEM((ic, b), jnp.int32),
            ]
            + [pltpu.VMEM((b, d), jnp.float32)] * NBUF
            + [pltpu.VMEM_SHARED((n, d), jnp.float32)]
            + [pltpu.SemaphoreType.DMA] * (2 * ic + 2 * NBUF)
        ),
    )
    def seg_kernel(x_hbm, src_hbm, dst_hbm, zeros_hbm, out_hbm,
                   src_ring, dst_ring, *rest):
        bufs = rest[:NBUF]
        acc_sh = rest[NBUF]
        isrc = rest[NBUF + 1:NBUF + 1 + ic]
        idst = rest[NBUF + 1 + ic:NBUF + 1 + 2 * ic]
        gsem = rest[NBUF + 1 + 2 * ic:NBUF + 1 + 2 * ic + NBUF]
        ssem = rest[NBUF + 1 + 2 * ic + NBUF:]
        cid = lax.axis_index("c")
        sid = lax.axis_index("s")
        wid = sid * NC + cid
        my_rows = pl.ds(sid * rows_per_tile, rows_per_tile)
        tail_rows = pl.ds(NS * rows_per_tile, tail)
        for t in range(ic):
            pltpu.async_copy(src_hbm.at[wid, t], src_ring.at[t], isrc[t])
        for t in range(ic - 1):
            pltpu.async_copy(dst_hbm.at[wid, t], dst_ring.at[t], idst[t])
        for s in range(2):
            pltpu.make_async_copy(
                src_hbm.at[wid, s], src_ring.at[s], isrc[s]).wait()
            pltpu.async_copy(x_hbm.at[src_ring.at[s]], bufs[s], gsem[s])
        pltpu.sync_copy(zeros_hbm.at[my_rows], acc_sh.at[my_rows])
        if tail:
            @pl.when(sid == NS - 1)
            def _zero_tail():
                pltpu.sync_copy(zeros_hbm.at[tail_rows], acc_sh.at[tail_rows])
        plsc.subcore_barrier()

        def emit_chunk(c, k):
            s = k % NBUF
            s2 = (k + 2) % NBUF
            k2 = (k + 2) % ic
            pltpu.make_async_copy(
                x_hbm.at[src_ring.at[k]], bufs[s], gsem[s]).wait()
            pltpu.make_async_copy(
                dst_hbm.at[wid, c], dst_ring.at[k], idst[k]).wait()
            pltpu.async_copy(
                bufs[s], acc_sh.at[dst_ring.at[k]], ssem[s], add=True)

            @pl.when(c >= 1)
            def _drain_prev():
                pltpu.make_async_copy(
                    bufs[s2], acc_sh.at[dst_ring.at[k]], ssem[s2]).wait()

            @pl.when(c + (ic - 1) < c_chunks)
            def _refill_dst():
                pltpu.async_copy(dst_hbm.at[wid, c + (ic - 1)],
                                 dst_ring.at[(k + ic - 1) % ic],
                                 idst[(k + ic - 1) % ic])

            @pl.when(c + ic < c_chunks)
            def _refill_src():
                pltpu.async_copy(
                    src_hbm.at[wid, c + ic], src_ring.at[k], isrc[k])

            @pl.when(c + 2 < c_chunks)
            def _refill_rows():
                pltpu.make_async_copy(
                    src_hbm.at[wid, c + 2], src_ring.at[k2], isrc[k2]).wait()
                pltpu.async_copy(x_hbm.at[src_ring.at[k2]], bufs[s2], gsem[s2])

        def body(g, carry):
            for k in range(ic):
                emit_chunk(g * ic + k, k)
            return carry

        n_groups = c_chunks // ic
        lax.fori_loop(0, n_groups, body, 0)
        for k in range(c_chunks - n_groups * ic):
            emit_chunk(n_groups * ic + k, k)
        s_last = (c_chunks - 1) % NBUF
        pltpu.make_async_copy(
            bufs[s_last], acc_sh.at[dst_ring.at[0]], ssem[s_last]).wait()
        plsc.subcore_barrier()
        pltpu.sync_copy(acc_sh.at[my_rows], out_hbm.at[cid, my_rows])
        if tail:
            @pl.when(sid == NS - 1)
            def _out_tail():
                pltpu.sync_copy(acc_sh.at[tail_rows], out_hbm.at[cid, tail_rows])

    return seg_kernel(x, src3, dst3, zeros)


def _dense_tc(partials, w, bias, relu):
    _, n, d = partials.shape
    d_out = w.shape[1]
    bn = 5000

    def body(p_ref, w_ref, b_ref, o_ref):
        a = p_ref[0] + p_ref[1]
        y = jnp.dot(a, w_ref[...], preferred_element_type=jnp.float32)
        y = y + b_ref[...]
        if relu:
            y = jnp.maximum(y, 0.0)
        o_ref[...] = y

    return pl.pallas_call(
        body,
        grid=(n // bn,),
        in_specs=[
            pl.BlockSpec((NC, bn, d), lambda i: (0, i, 0)),
            pl.BlockSpec((d, d_out), lambda i: (0, 0)),
            pl.BlockSpec((1, d_out), lambda i: (0, 0)),
        ],
        out_specs=pl.BlockSpec((bn, d_out), lambda i: (i, 0)),
        out_shape=jax.ShapeDtypeStruct((n, d_out), jnp.float32),
    )(partials, w, bias.reshape(1, -1))


def kernel(features, edge_index, W0, b0, W1, b1):
    n, d_in = features.shape
    e = edge_index.shape[1]
    e_per_w = e // NW
    b = 100
    c_chunks = e_per_w // b
    src3 = edge_index[0].reshape(NW, c_chunks, b)
    dst3 = edge_index[1].reshape(NW, c_chunks, b)
    zeros = jnp.zeros((n, d_in), jnp.float32)

    p0 = _segment_sum_sc(features, src3, dst3, zeros)
    h = _dense_tc(p0, W0, b0, relu=True)
    p1 = _segment_sum_sc(h, src3, dst3, zeros)
    return _dense_tc(p1, W1, b1, relu=False)
